# baseline (speedup 1.0000x reference)
import functools

import jax
import jax.numpy as jnp
from jax import lax
from jax.experimental import pallas as pl
from jax.experimental.pallas import tpu as pltpu
from jax.experimental.pallas import tpu_sc as plsc

N = 10000
NP = 10240
E = 320000
D = 128

NC = 2
NS = 16
NW = NC * NS

B = 80
K = 128
EW = B * K
EP = NW * EW
S = NP // NS
NBUF = 2

_mesh = plsc.VectorSubcoreMesh(core_axis_name="c", subcore_axis_name="s")
_sc_params = pltpu.CompilerParams(use_tc_tiling_on_sc=False)


def _fill_rows(buf, nrows, ncols, vec):
  def body(i, _):
    for j in range(ncols // 16):
      buf[i, pl.ds(j * 16, 16)] = vec
    return 0

  lax.fori_loop(0, nrows, body, 0)


def _stage_dst(dst_hbm, didx, base, isem):
  def start(j, _):
    pltpu.make_async_copy(dst_hbm.at[pl.ds(base + j * B, B)], didx.at[j],
                          isem).start()
    return 0

  lax.fori_loop(0, K, start, 0)

  def drain(j, _):
    pltpu.make_async_copy(dst_hbm.at[pl.ds(base + j * B, B)], didx.at[j],
                          isem).wait()
    return 0

  lax.fori_loop(0, K, drain, 0)


@functools.partial(
    pl.kernel,
    out_type=jax.ShapeDtypeStruct((NC * NP, D), jnp.float32),
    mesh=_mesh,
    scratch_types=[
        pltpu.VMEM((K, B), jnp.int32),
        pltpu.VMEM((B, 16), jnp.float32),
        pltpu.VMEM((S, 16), jnp.float32),
        pltpu.VMEM((S, D), jnp.float32),
        pltpu.VMEM_SHARED((NP, 16), jnp.float32),
        pltpu.SemaphoreType.DMA,
        pltpu.SemaphoreType.DMA,
    ],
    compiler_params=_sc_params,
)
def _deg_kernel(dst_hbm, out_hbm, didx, ones_v, buf16, buf128, acc, isem,
                ssem):
  c = lax.axis_index("c")
  s = lax.axis_index("s")
  wid = c * NS + s

  _fill_rows(ones_v, B, 16, jnp.ones((16,), jnp.float32))
  _fill_rows(buf16, S, 16, jnp.zeros((16,), jnp.float32))
  _fill_rows(buf128, S, D, jnp.zeros((16,), jnp.float32))

  pltpu.sync_copy(buf16, acc.at[pl.ds(s * S, S)])

  _stage_dst(dst_hbm, didx, wid * EW, isem)
  plsc.subcore_barrier()

  def fire(j, _):
    pltpu.async_copy(ones_v, acc.at[didx.at[j]], ssem, add=True)
    return 0

  lax.fori_loop(0, K, fire, 0)

  def drain(j, _):
    pltpu.make_async_copy(ones_v, acc.at[didx.at[j]], ssem).wait()
    return 0

  lax.fori_loop(0, K, drain, 0)
  plsc.subcore_barrier()

  pltpu.sync_copy(acc.at[pl.ds(s * S, S)], buf16)

  def widen(i, _):
    buf128[i, pl.ds(0, 16)] = buf16[i, :]
    return 0

  lax.fori_loop(0, S, widen, 0)
  pltpu.sync_copy(buf128, out_hbm.at[pl.ds(c * NP + s * S, S)])


def _make_agg(pipelined):
  @functools.partial(
      pl.kernel,
      out_type=jax.ShapeDtypeStruct((NC * NP, D), jnp.float32),
      mesh=_mesh,
      scratch_types=[
          pltpu.VMEM((EW,), jnp.int32),
          pltpu.VMEM((K, B), jnp.int32),
          pltpu.VMEM((NBUF, B, D), jnp.float32),
          pltpu.VMEM_SHARED((NP, D), jnp.float32),
          pltpu.SemaphoreType.DMA,
          pltpu.SemaphoreType.DMA,
          [pltpu.SemaphoreType.DMA] * NBUF,
      ],
      compiler_params=_sc_params,
  )
  def _agg_kernel(y_hbm, src_hbm, dst_hbm, out_hbm, sidx, didx, rows, acc,
                  isem, gsem, ssems):
    c = lax.axis_index("c")
    s = lax.axis_index("s")
    wid = c * NS + s

    _fill_rows(rows.at[0], B, D, jnp.zeros((16,), jnp.float32))
    for k in range(S // B):
      pltpu.sync_copy(rows.at[0], acc.at[pl.ds(s * S + k * B, B)])

    pltpu.sync_copy(src_hbm.at[pl.ds(wid * EW, EW)], sidx)
    _stage_dst(dst_hbm, didx, wid * EW, isem)
    plsc.subcore_barrier()

    if pipelined:
      def outer(i, _):
        for b in range(NBUF):
          j = i * NBUF + b

          @pl.when(i > 0)
          def _():
            pltpu.make_async_copy(rows.at[b], acc.at[didx.at[j - NBUF]],
                                  ssems[b]).wait()

          pltpu.async_copy(y_hbm.at[sidx.at[pl.ds(j * B, B)]], rows.at[b],
                           gsem).wait()
          pltpu.async_copy(rows.at[b], acc.at[didx.at[j]], ssems[b],
                           add=True)
        return 0

      lax.fori_loop(0, K // NBUF, outer, 0)
      for b in range(NBUF):
        pltpu.make_async_copy(rows.at[b], acc.at[didx.at[K - NBUF + b]],
                              ssems[b]).wait()
    else:
      def body(j, _):
        pltpu.async_copy(y_hbm.at[sidx.at[pl.ds(j * B, B)]], rows.at[0],
                         gsem).wait()
        pltpu.sync_copy(rows.at[0], acc.at[didx.at[j]], add=True)
        return 0

      lax.fori_loop(0, K, body, 0)
    plsc.subcore_barrier()

    pltpu.sync_copy(acc.at[pl.ds(s * S, S)],
                    out_hbm.at[pl.ds(c * NP + s * S, S)])

  return _agg_kernel


_agg_sync = _make_agg(False)
_agg_pipe = _make_agg(True)


RB = 1280
_GRID = (NP // RB,)
_NB = NP // RB


def _row_spec(cols, off=0):
  return pl.BlockSpec((RB, cols), lambda i, o=off: (i + o, 0))


def _full_spec(r, c):
  return pl.BlockSpec((r, c), lambda i: (0, 0))


def _deg_terms(d0, d1):
  deg = 1.0 + d0[:, 0:1] + d1[:, 0:1]
  dis = lax.rsqrt(deg)
  return dis, 1.0 / deg


def _tc1_body(x_ref, w1_ref, d0_ref, d1_ref, xw_ref, y_ref):
  dis, _ = _deg_terms(d0_ref[...], d1_ref[...])
  xw = jnp.dot(x_ref[...], w1_ref[...], preferred_element_type=jnp.float32)
  xw_ref[...] = xw
  y_ref[...] = xw * dis


def _tc2_body(p0_ref, p1_ref, xw1_ref, d0_ref, d1_ref, b1_ref, w2_ref,
              h_ref, xw2_ref, y2_ref):
  dis, deginv = _deg_terms(d0_ref[...], d1_ref[...])
  pre = ((p0_ref[...] + p1_ref[...]) * dis + xw1_ref[...] * deginv
         + b1_ref[...])
  h = jnp.maximum(pre, 0.0)
  h_ref[...] = h
  xw2 = jnp.dot(h, w2_ref[...], preferred_element_type=jnp.float32)
  xw2_ref[...] = xw2
  y2_ref[...] = xw2 * dis


def _tc3_body(q0_ref, q1_ref, xw2_ref, d0_ref, d1_ref, b2_ref, out_ref):
  dis, deginv = _deg_terms(d0_ref[...], d1_ref[...])
  out_ref[...] = ((q0_ref[...] + q1_ref[...]) * dis
                  + xw2_ref[...] * deginv + b2_ref[...])


_tc1 = pl.pallas_call(
    _tc1_body,
    grid=_GRID,
    in_specs=[_row_spec(D), _full_spec(D, D), _row_spec(D), _row_spec(D, _NB)],
    out_specs=[_row_spec(D), _row_spec(D)],
    out_shape=[jax.ShapeDtypeStruct((NP, D), jnp.float32)] * 2,
)

_tc2 = pl.pallas_call(
    _tc2_body,
    grid=_GRID,
    in_specs=[_row_spec(D), _row_spec(D, _NB), _row_spec(D), _row_spec(D),
              _row_spec(D, _NB), _full_spec(1, D), _full_spec(D, D)],
    out_specs=[_row_spec(D), _row_spec(D), _row_spec(D)],
    out_shape=[jax.ShapeDtypeStruct((NP, D), jnp.float32)] * 3,
)

_tc3 = pl.pallas_call(
    _tc3_body,
    grid=_GRID,
    in_specs=[_row_spec(D), _row_spec(D, _NB), _row_spec(D), _row_spec(D),
              _row_spec(D, _NB), _full_spec(1, D)],
    out_specs=_row_spec(D),
    out_shape=jax.ShapeDtypeStruct((NP, D), jnp.float32),
)


def kernel(x, edge_index, W1, b1, W2, b2):
  src = jnp.concatenate([edge_index[0],
                         jnp.zeros((EP - E,), jnp.int32)])
  pad_dst = N + jnp.arange(EP - E, dtype=jnp.int32) % (NP - N)
  dst = jnp.concatenate([edge_index[1], pad_dst])
  x_p = jnp.concatenate([x, jnp.zeros((NP - N, D), jnp.float32)], axis=0)

  dp = _deg_kernel(dst)
  xw1, y1 = _tc1(x_p, W1, dp, dp)
  p = _agg_sync(y1, src, dst)
  h, xw2, y2 = _tc2(p, p, xw1, dp, dp, b1.reshape(1, D), W2)
  q = _agg_pipe(y2, src, dst)
  logits = _tc3(q, q, xw2, dp, dp, b2.reshape(1, D))
  return (h[:N], logits[:N])


# speedup vs baseline: 2.5531x; 2.5531x over previous
import functools

import jax
import jax.numpy as jnp
from jax import lax
from jax.experimental import pallas as pl
from jax.experimental.pallas import tpu as pltpu
from jax.experimental.pallas import tpu_sc as plsc

N = 10000
NP = 10240
E = 320000
D = 128

NC = 2
NS = 16
NW = NC * NS

B = 80
K = 128
EW = B * K
EP = NW * EW
S = NP // NS
NBUF = 2

_mesh = plsc.VectorSubcoreMesh(core_axis_name="c", subcore_axis_name="s")
_sc_params = pltpu.CompilerParams(use_tc_tiling_on_sc=False)


def _fill_rows(buf, nrows, ncols, vec):
  def body(i, _):
    for j in range(ncols // 16):
      buf[i, pl.ds(j * 16, 16)] = vec
    return 0

  lax.fori_loop(0, nrows, body, 0)


def _stage_dst(dst_hbm, didx, base, isem):
  def start(j, _):
    pltpu.make_async_copy(dst_hbm.at[pl.ds(base + j * B, B)], didx.at[j],
                          isem).start()
    return 0

  lax.fori_loop(0, K, start, 0)

  def drain(j, _):
    pltpu.make_async_copy(dst_hbm.at[pl.ds(base + j * B, B)], didx.at[j],
                          isem).wait()
    return 0

  lax.fori_loop(0, K, drain, 0)


@functools.partial(
    pl.kernel,
    out_type=jax.ShapeDtypeStruct((NC * NP, D), jnp.float32),
    mesh=_mesh,
    scratch_types=[
        pltpu.VMEM((K, B), jnp.int32),
        pltpu.VMEM((B, 16), jnp.float32),
        pltpu.VMEM((S, 16), jnp.float32),
        pltpu.VMEM((S, D), jnp.float32),
        pltpu.VMEM_SHARED((NP, 16), jnp.float32),
        pltpu.SemaphoreType.DMA,
        pltpu.SemaphoreType.DMA,
    ],
    compiler_params=_sc_params,
)
def _deg_kernel(dst_hbm, out_hbm, didx, ones_v, buf16, buf128, acc, isem,
                ssem):
  c = lax.axis_index("c")
  s = lax.axis_index("s")
  wid = c * NS + s

  _fill_rows(ones_v, B, 16, jnp.ones((16,), jnp.float32))
  _fill_rows(buf16, S, 16, jnp.zeros((16,), jnp.float32))
  _fill_rows(buf128, S, D, jnp.zeros((16,), jnp.float32))

  pltpu.sync_copy(buf16, acc.at[pl.ds(s * S, S)])

  _stage_dst(dst_hbm, didx, wid * EW, isem)
  plsc.subcore_barrier()

  def fire(j, _):
    pltpu.async_copy(ones_v, acc.at[didx.at[j]], ssem, add=True)
    return 0

  lax.fori_loop(0, K, fire, 0)

  def drain(j, _):
    pltpu.make_async_copy(ones_v, acc.at[didx.at[j]], ssem).wait()
    return 0

  lax.fori_loop(0, K, drain, 0)
  plsc.subcore_barrier()

  pltpu.sync_copy(acc.at[pl.ds(s * S, S)], buf16)

  def widen(i, _):
    buf128[i, pl.ds(0, 16)] = buf16[i, :]
    return 0

  lax.fori_loop(0, S, widen, 0)
  pltpu.sync_copy(buf128, out_hbm.at[pl.ds(c * NP + s * S, S)])


def _make_agg(pipelined):
  @functools.partial(
      pl.kernel,
      out_type=jax.ShapeDtypeStruct((NC * NP, D), jnp.float32),
      mesh=_mesh,
      scratch_types=[
          pltpu.VMEM((EW,), jnp.int32),
          pltpu.VMEM((K, B), jnp.int32),
          pltpu.VMEM((NBUF, B, D), jnp.float32),
          pltpu.VMEM_SHARED((NP, D), jnp.float32),
          pltpu.SemaphoreType.DMA,
          pltpu.SemaphoreType.DMA,
          [pltpu.SemaphoreType.DMA] * NBUF,
      ],
      compiler_params=_sc_params,
  )
  def _agg_kernel(y_hbm, src_hbm, dst_hbm, out_hbm, sidx, didx, rows, acc,
                  isem, gsem, ssems):
    c = lax.axis_index("c")
    s = lax.axis_index("s")
    wid = c * NS + s

    _fill_rows(rows.at[0], B, D, jnp.zeros((16,), jnp.float32))
    for k in range(S // B):
      pltpu.sync_copy(rows.at[0], acc.at[pl.ds(s * S + k * B, B)])

    pltpu.sync_copy(src_hbm.at[pl.ds(wid * EW, EW)], sidx)
    _stage_dst(dst_hbm, didx, wid * EW, isem)
    plsc.subcore_barrier()

    if pipelined:
      def outer(i, _):
        for b in range(NBUF):
          j = i * NBUF + b

          @pl.when(i > 0)
          def _():
            pltpu.make_async_copy(rows.at[b], acc.at[didx.at[j - NBUF]],
                                  ssems[b]).wait()

          pltpu.async_copy(y_hbm.at[sidx.at[pl.ds(j * B, B)]], rows.at[b],
                           gsem).wait()
          pltpu.async_copy(rows.at[b], acc.at[didx.at[j]], ssems[b],
                           add=True)
        return 0

      lax.fori_loop(0, K // NBUF, outer, 0)
      for b in range(NBUF):
        pltpu.make_async_copy(rows.at[b], acc.at[didx.at[K - NBUF + b]],
                              ssems[b]).wait()
    else:
      def body(j, _):
        pltpu.async_copy(y_hbm.at[sidx.at[pl.ds(j * B, B)]], rows.at[0],
                         gsem).wait()
        pltpu.sync_copy(rows.at[0], acc.at[didx.at[j]], add=True)
        return 0

      lax.fori_loop(0, K, body, 0)
    plsc.subcore_barrier()

    pltpu.sync_copy(acc.at[pl.ds(s * S, S)],
                    out_hbm.at[pl.ds(c * NP + s * S, S)])

  return _agg_kernel


_agg_sync = _make_agg(False)
_agg_pipe = _make_agg(True)


RB = 1280
_GRID = (NP // RB,)
_NB = NP // RB


def _row_spec(cols, off=0):
  return pl.BlockSpec((RB, cols), lambda i, o=off: (i + o, 0))


def _full_spec(r, c):
  return pl.BlockSpec((r, c), lambda i: (0, 0))


def _deg_terms(d0, d1):
  deg = 1.0 + d0[:, 0:1] + d1[:, 0:1]
  dis = lax.rsqrt(deg)
  return dis, 1.0 / deg


def _tc1_body(x_ref, w1_ref, d0_ref, d1_ref, xw_ref, y_ref):
  dis, _ = _deg_terms(d0_ref[...], d1_ref[...])
  xw = jnp.dot(x_ref[...], w1_ref[...], preferred_element_type=jnp.float32)
  xw_ref[...] = xw
  y_ref[...] = xw * dis


def _tc2_body(p0_ref, p1_ref, xw1_ref, d0_ref, d1_ref, b1_ref, w2_ref,
              h_ref, xw2_ref, y2_ref):
  dis, deginv = _deg_terms(d0_ref[...], d1_ref[...])
  pre = ((p0_ref[...] + p1_ref[...]) * dis + xw1_ref[...] * deginv
         + b1_ref[...])
  h = jnp.maximum(pre, 0.0)
  h_ref[...] = h
  xw2 = jnp.dot(h, w2_ref[...], preferred_element_type=jnp.float32)
  xw2_ref[...] = xw2
  y2_ref[...] = xw2 * dis


def _tc3_body(q0_ref, q1_ref, xw2_ref, d0_ref, d1_ref, b2_ref, out_ref):
  dis, deginv = _deg_terms(d0_ref[...], d1_ref[...])
  out_ref[...] = ((q0_ref[...] + q1_ref[...]) * dis
                  + xw2_ref[...] * deginv + b2_ref[...])


_tc1 = pl.pallas_call(
    _tc1_body,
    grid=_GRID,
    in_specs=[_row_spec(D), _full_spec(D, D), _row_spec(D), _row_spec(D, _NB)],
    out_specs=[_row_spec(D), _row_spec(D)],
    out_shape=[jax.ShapeDtypeStruct((NP, D), jnp.float32)] * 2,
)

_tc2 = pl.pallas_call(
    _tc2_body,
    grid=_GRID,
    in_specs=[_row_spec(D), _row_spec(D, _NB), _row_spec(D), _row_spec(D),
              _row_spec(D, _NB), _full_spec(1, D), _full_spec(D, D)],
    out_specs=[_row_spec(D), _row_spec(D), _row_spec(D)],
    out_shape=[jax.ShapeDtypeStruct((NP, D), jnp.float32)] * 3,
)

_tc3 = pl.pallas_call(
    _tc3_body,
    grid=_GRID,
    in_specs=[_row_spec(D), _row_spec(D, _NB), _row_spec(D), _row_spec(D),
              _row_spec(D, _NB), _full_spec(1, D)],
    out_specs=_row_spec(D),
    out_shape=jax.ShapeDtypeStruct((NP, D), jnp.float32),
)


def kernel(x, edge_index, W1, b1, W2, b2):
  pad_iota = jnp.arange(EP - E, dtype=jnp.int32)
  src = jnp.concatenate([edge_index[0], pad_iota % N])
  dst = jnp.concatenate([edge_index[1], N + pad_iota % (NP - N)])
  x_p = jnp.concatenate([x, jnp.zeros((NP - N, D), jnp.float32)], axis=0)

  dp = _deg_kernel(dst)
  xw1, y1 = _tc1(x_p, W1, dp, dp)
  p = _agg_sync(y1, src, dst)
  h, xw2, y2 = _tc2(p, p, xw1, dp, dp, b1.reshape(1, D), W2)
  q = _agg_pipe(y2, src, dst)
  logits = _tc3(q, q, xw2, dp, dp, b2.reshape(1, D))
  return (h[:N], logits[:N])


# speedup vs baseline: 2.8721x; 1.1249x over previous
import functools

import jax
import jax.numpy as jnp
from jax import lax
from jax.experimental import pallas as pl
from jax.experimental.pallas import tpu as pltpu
from jax.experimental.pallas import tpu_sc as plsc

N = 10000
NP = 10240
E = 320000
D = 128

NC = 2
NS = 16
NW = NC * NS

B = 80
K = 128
EW = B * K
EP = NW * EW
S = NP // NS
NBUF = 2

_mesh = plsc.VectorSubcoreMesh(core_axis_name="c", subcore_axis_name="s")
_sc_params = pltpu.CompilerParams(use_tc_tiling_on_sc=False)


def _fill_rows(buf, nrows, ncols, vec):
  def body(i, _):
    for j in range(ncols // 16):
      buf[i, pl.ds(j * 16, 16)] = vec
    return 0

  lax.fori_loop(0, nrows, body, 0)


def _stage_dst(dst_hbm, didx, base, isem):
  def start(j, _):
    pltpu.make_async_copy(dst_hbm.at[pl.ds(base + j * B, B)], didx.at[j],
                          isem).start()
    return 0

  lax.fori_loop(0, K, start, 0)

  def drain(j, _):
    pltpu.make_async_copy(dst_hbm.at[pl.ds(base + j * B, B)], didx.at[j],
                          isem).wait()
    return 0

  lax.fori_loop(0, K, drain, 0)


@functools.partial(
    pl.kernel,
    out_type=jax.ShapeDtypeStruct((NC * NP, D), jnp.float32),
    mesh=_mesh,
    scratch_types=[
        pltpu.VMEM((K, B), jnp.int32),
        pltpu.VMEM((B, 16), jnp.float32),
        pltpu.VMEM((S, 16), jnp.float32),
        pltpu.VMEM((S, D), jnp.float32),
        pltpu.VMEM_SHARED((NP, 16), jnp.float32),
        pltpu.SemaphoreType.DMA,
        pltpu.SemaphoreType.DMA,
    ],
    compiler_params=_sc_params,
)
def _deg_kernel(dst_hbm, out_hbm, didx, ones_v, buf16, buf128, acc, isem,
                ssem):
  c = lax.axis_index("c")
  s = lax.axis_index("s")
  wid = c * NS + s

  _fill_rows(ones_v, B, 16, jnp.ones((16,), jnp.float32))
  _fill_rows(buf16, S, 16, jnp.zeros((16,), jnp.float32))
  _fill_rows(buf128, S, D, jnp.zeros((16,), jnp.float32))

  pltpu.sync_copy(buf16, acc.at[pl.ds(s * S, S)])

  _stage_dst(dst_hbm, didx, wid * EW, isem)
  plsc.subcore_barrier()

  def fire(j, _):
    pltpu.async_copy(ones_v, acc.at[didx.at[j]], ssem, add=True)
    return 0

  lax.fori_loop(0, K, fire, 0)

  def drain(j, _):
    pltpu.make_async_copy(ones_v, acc.at[didx.at[j]], ssem).wait()
    return 0

  lax.fori_loop(0, K, drain, 0)
  plsc.subcore_barrier()

  pltpu.sync_copy(acc.at[pl.ds(s * S, S)], buf16)

  def widen(i, _):
    buf128[i, pl.ds(0, 16)] = buf16[i, :]
    return 0

  lax.fori_loop(0, S, widen, 0)
  pltpu.sync_copy(buf128, out_hbm.at[pl.ds(c * NP + s * S, S)])


def _make_agg(pipelined):
  @functools.partial(
      pl.kernel,
      out_type=jax.ShapeDtypeStruct((NC * NP, D), jnp.float32),
      mesh=_mesh,
      scratch_types=[
          pltpu.VMEM((EW,), jnp.int32),
          pltpu.VMEM((K, B), jnp.int32),
          pltpu.VMEM((NBUF, B, D), jnp.float32),
          pltpu.VMEM_SHARED((NP, D), jnp.float32),
          pltpu.SemaphoreType.DMA,
          pltpu.SemaphoreType.DMA,
          [pltpu.SemaphoreType.DMA] * NBUF,
      ],
      compiler_params=_sc_params,
  )
  def _agg_kernel(y_hbm, src_hbm, dst_hbm, out_hbm, sidx, didx, rows, acc,
                  isem, gsem, ssems):
    c = lax.axis_index("c")
    s = lax.axis_index("s")
    wid = c * NS + s

    _fill_rows(rows.at[0], B, D, jnp.zeros((16,), jnp.float32))
    for k in range(S // B):
      pltpu.sync_copy(rows.at[0], acc.at[pl.ds(s * S + k * B, B)])

    pltpu.sync_copy(src_hbm.at[pl.ds(wid * EW, EW)], sidx)
    _stage_dst(dst_hbm, didx, wid * EW, isem)
    plsc.subcore_barrier()

    if pipelined:
      def outer(i, _):
        for b in range(NBUF):
          j = i * NBUF + b

          @pl.when(i > 0)
          def _():
            pltpu.make_async_copy(rows.at[b], acc.at[didx.at[j - NBUF]],
                                  ssems[b]).wait()

          pltpu.async_copy(y_hbm.at[sidx.at[pl.ds(j * B, B)]], rows.at[b],
                           gsem).wait()
          pltpu.async_copy(rows.at[b], acc.at[didx.at[j]], ssems[b],
                           add=True)
        return 0

      lax.fori_loop(0, K // NBUF, outer, 0)
      for b in range(NBUF):
        pltpu.make_async_copy(rows.at[b], acc.at[didx.at[K - NBUF + b]],
                              ssems[b]).wait()
    else:
      def body(j, _):
        pltpu.async_copy(y_hbm.at[sidx.at[pl.ds(j * B, B)]], rows.at[0],
                         gsem).wait()
        pltpu.sync_copy(rows.at[0], acc.at[didx.at[j]], add=True)
        return 0

      lax.fori_loop(0, K, body, 0)
    plsc.subcore_barrier()

    pltpu.sync_copy(acc.at[pl.ds(s * S, S)],
                    out_hbm.at[pl.ds(c * NP + s * S, S)])

  return _agg_kernel


_agg_sync = _make_agg(False)
_agg_pipe = _make_agg(True)


RB = 1280
_GRID = (NP // RB,)
_NB = NP // RB


def _row_spec(cols, off=0):
  return pl.BlockSpec((RB, cols), lambda i, o=off: (i + o, 0))


def _full_spec(r, c):
  return pl.BlockSpec((r, c), lambda i: (0, 0))


def _deg_terms(d0, d1):
  deg = 1.0 + d0[:, 0:1] + d1[:, 0:1]
  dis = lax.rsqrt(deg)
  return dis, 1.0 / deg


def _tc1_body(x_ref, w1_ref, d0_ref, d1_ref, xw_ref, y_ref):
  dis, _ = _deg_terms(d0_ref[...], d1_ref[...])
  xw = jnp.dot(x_ref[...], w1_ref[...], preferred_element_type=jnp.float32)
  xw_ref[...] = xw
  y_ref[...] = xw * dis


def _tc2_body(p0_ref, p1_ref, xw1_ref, d0_ref, d1_ref, b1_ref, w2_ref,
              h_ref, xw2_ref, y2_ref):
  dis, deginv = _deg_terms(d0_ref[...], d1_ref[...])
  pre = ((p0_ref[...] + p1_ref[...]) * dis + xw1_ref[...] * deginv
         + b1_ref[...])
  h = jnp.maximum(pre, 0.0)
  h_ref[...] = h
  xw2 = jnp.dot(h, w2_ref[...], preferred_element_type=jnp.float32)
  xw2_ref[...] = xw2
  y2_ref[...] = xw2 * dis


def _tc3_body(q0_ref, q1_ref, xw2_ref, d0_ref, d1_ref, b2_ref, out_ref):
  dis, deginv = _deg_terms(d0_ref[...], d1_ref[...])
  out_ref[...] = ((q0_ref[...] + q1_ref[...]) * dis
                  + xw2_ref[...] * deginv + b2_ref[...])


_tc1 = pl.pallas_call(
    _tc1_body,
    grid=_GRID,
    in_specs=[_row_spec(D), _full_spec(D, D), _row_spec(D), _row_spec(D, _NB)],
    out_specs=[_row_spec(D), _row_spec(D)],
    out_shape=[jax.ShapeDtypeStruct((NP, D), jnp.float32)] * 2,
)

_tc2 = pl.pallas_call(
    _tc2_body,
    grid=_GRID,
    in_specs=[_row_spec(D), _row_spec(D, _NB), _row_spec(D), _row_spec(D),
              _row_spec(D, _NB), _full_spec(1, D), _full_spec(D, D)],
    out_specs=[_row_spec(D), _row_spec(D), _row_spec(D)],
    out_shape=[jax.ShapeDtypeStruct((NP, D), jnp.float32)] * 3,
)

_tc3 = pl.pallas_call(
    _tc3_body,
    grid=_GRID,
    in_specs=[_row_spec(D), _row_spec(D, _NB), _row_spec(D), _row_spec(D),
              _row_spec(D, _NB), _full_spec(1, D)],
    out_specs=_row_spec(D),
    out_shape=jax.ShapeDtypeStruct((NP, D), jnp.float32),
)


def kernel(x, edge_index, W1, b1, W2, b2):
  pad_iota = jnp.arange(EP - E, dtype=jnp.int32)
  src = jnp.concatenate([edge_index[0], pad_iota % N])
  dst = jnp.concatenate([edge_index[1], N + pad_iota % (NP - N)])
  x_p = jnp.concatenate([x, jnp.zeros((NP - N, D), jnp.float32)], axis=0)

  dp = _deg_kernel(dst)
  xw1, y1 = _tc1(x_p, W1, dp, dp)
  p = _agg_pipe(y1, src, dst)
  h, xw2, y2 = _tc2(p, p, xw1, dp, dp, b1.reshape(1, D), W2)
  q = _agg_pipe(y2, src, dst)
  logits = _tc3(q, q, xw2, dp, dp, b2.reshape(1, D))
  return (h[:N], logits[:N])


# speedup vs baseline: 3.5145x; 1.2237x over previous
import functools

import jax
import jax.numpy as jnp
from jax import lax
from jax.experimental import pallas as pl
from jax.experimental.pallas import tpu as pltpu
from jax.experimental.pallas import tpu_sc as plsc

N = 10000
NP = 10240
E = 320000
D = 128

NC = 2
NS = 16
NW = NC * NS

B = 80
K = 128
EW = B * K
EP = NW * EW
S = NP // NS
NBUF = 2

_mesh = plsc.VectorSubcoreMesh(core_axis_name="c", subcore_axis_name="s")
_sc_params = pltpu.CompilerParams(use_tc_tiling_on_sc=False)


def _fill_rows(buf, nrows, ncols, vec):
  def body(i, _):
    for j in range(ncols // 16):
      buf[i, pl.ds(j * 16, 16)] = vec
    return 0

  lax.fori_loop(0, nrows, body, 0)


def _stage_dst(dst_hbm, didx, base, isem):
  def start(j, _):
    pltpu.make_async_copy(dst_hbm.at[pl.ds(base + j * B, B)], didx.at[j],
                          isem).start()
    return 0

  lax.fori_loop(0, K, start, 0)

  def drain(j, _):
    pltpu.make_async_copy(dst_hbm.at[pl.ds(base + j * B, B)], didx.at[j],
                          isem).wait()
    return 0

  lax.fori_loop(0, K, drain, 0)


@functools.partial(
    pl.kernel,
    out_type=jax.ShapeDtypeStruct((NC * NP, D), jnp.float32),
    mesh=_mesh,
    scratch_types=[
        pltpu.VMEM((K, B), jnp.int32),
        pltpu.VMEM((B, 16), jnp.float32),
        pltpu.VMEM((S, 16), jnp.float32),
        pltpu.VMEM((S, D), jnp.float32),
        pltpu.VMEM_SHARED((NP, 16), jnp.float32),
        pltpu.SemaphoreType.DMA,
        pltpu.SemaphoreType.DMA,
    ],
    compiler_params=_sc_params,
)
def _deg_kernel(dst_hbm, out_hbm, didx, ones_v, buf16, buf128, acc, isem,
                ssem):
  c = lax.axis_index("c")
  s = lax.axis_index("s")
  wid = c * NS + s

  _fill_rows(ones_v, B, 16, jnp.ones((16,), jnp.float32))
  _fill_rows(buf16, S, 16, jnp.zeros((16,), jnp.float32))
  _fill_rows(buf128, S, D, jnp.zeros((16,), jnp.float32))

  pltpu.sync_copy(buf16, acc.at[pl.ds(s * S, S)])

  _stage_dst(dst_hbm, didx, wid * EW, isem)
  plsc.subcore_barrier()

  def fire(j, _):
    pltpu.async_copy(ones_v, acc.at[didx.at[j]], ssem, add=True)
    return 0

  lax.fori_loop(0, K, fire, 0)

  def drain(j, _):
    pltpu.make_async_copy(ones_v, acc.at[didx.at[j]], ssem).wait()
    return 0

  lax.fori_loop(0, K, drain, 0)
  plsc.subcore_barrier()

  pltpu.sync_copy(acc.at[pl.ds(s * S, S)], buf16)

  def widen(i, _):
    buf128[i, pl.ds(0, 16)] = buf16[i, :]
    return 0

  lax.fori_loop(0, S, widen, 0)
  pltpu.sync_copy(buf128, out_hbm.at[pl.ds(c * NP + s * S, S)])


def _make_agg(pipelined):
  @functools.partial(
      pl.kernel,
      out_type=jax.ShapeDtypeStruct((NC * NP, D), jnp.float32),
      mesh=_mesh,
      scratch_types=[
          pltpu.VMEM((EW,), jnp.int32),
          pltpu.VMEM((K, B), jnp.int32),
          pltpu.VMEM((NBUF, B, D), jnp.float32),
          pltpu.VMEM_SHARED((NP, D), jnp.float32),
          pltpu.SemaphoreType.DMA,
          [pltpu.SemaphoreType.DMA] * NBUF,
          [pltpu.SemaphoreType.DMA] * NBUF,
      ],
      compiler_params=_sc_params,
  )
  def _agg_kernel(y_hbm, src_hbm, dst_hbm, out_hbm, sidx, didx, rows, acc,
                  isem, gsems, ssems):
    c = lax.axis_index("c")
    s = lax.axis_index("s")
    wid = c * NS + s

    _fill_rows(rows.at[0], B, D, jnp.zeros((16,), jnp.float32))
    for k in range(S // B):
      pltpu.sync_copy(rows.at[0], acc.at[pl.ds(s * S + k * B, B)])

    pltpu.sync_copy(src_hbm.at[pl.ds(wid * EW, EW)], sidx)
    _stage_dst(dst_hbm, didx, wid * EW, isem)
    plsc.subcore_barrier()

    if pipelined:
      pltpu.async_copy(y_hbm.at[sidx.at[pl.ds(0, B)]], rows.at[0],
                       gsems[0])

      def outer(i, _):
        for b in range(NBUF):
          j = i * NBUF + b
          nb = (b + 1) % NBUF

          @pl.when(j > 0)
          def _():
            pltpu.make_async_copy(rows.at[nb], acc.at[didx.at[j - 1]],
                                  ssems[nb]).wait()

          @pl.when(j < K - 1)
          def _():
            pltpu.async_copy(y_hbm.at[sidx.at[pl.ds((j + 1) * B, B)]],
                             rows.at[nb], gsems[nb])

          pltpu.make_async_copy(y_hbm.at[sidx.at[pl.ds(j * B, B)]],
                                rows.at[b], gsems[b]).wait()
          pltpu.async_copy(rows.at[b], acc.at[didx.at[j]], ssems[b],
                           add=True)
        return 0

      lax.fori_loop(0, K // NBUF, outer, 0)
      pltpu.make_async_copy(rows.at[(K - 1) % NBUF], acc.at[didx.at[K - 1]],
                            ssems[(K - 1) % NBUF]).wait()
    else:
      def body(j, _):
        pltpu.async_copy(y_hbm.at[sidx.at[pl.ds(j * B, B)]], rows.at[0],
                         gsems[0]).wait()
        pltpu.sync_copy(rows.at[0], acc.at[didx.at[j]], add=True)
        return 0

      lax.fori_loop(0, K, body, 0)
    plsc.subcore_barrier()

    pltpu.sync_copy(acc.at[pl.ds(s * S, S)],
                    out_hbm.at[pl.ds(c * NP + s * S, S)])

  return _agg_kernel


_agg_sync = _make_agg(False)
_agg_pipe = _make_agg(True)


RB = 1280
_GRID = (NP // RB,)
_NB = NP // RB


def _row_spec(cols, off=0):
  return pl.BlockSpec((RB, cols), lambda i, o=off: (i + o, 0))


def _full_spec(r, c):
  return pl.BlockSpec((r, c), lambda i: (0, 0))


def _deg_terms(d0, d1):
  deg = 1.0 + d0[:, 0:1] + d1[:, 0:1]
  dis = lax.rsqrt(deg)
  return dis, 1.0 / deg


def _tc1_body(x_ref, w1_ref, d0_ref, d1_ref, xw_ref, y_ref):
  dis, _ = _deg_terms(d0_ref[...], d1_ref[...])
  xw = jnp.dot(x_ref[...], w1_ref[...], preferred_element_type=jnp.float32)
  xw_ref[...] = xw
  y_ref[...] = xw * dis


def _tc2_body(p0_ref, p1_ref, xw1_ref, d0_ref, d1_ref, b1_ref, w2_ref,
              h_ref, xw2_ref, y2_ref):
  dis, deginv = _deg_terms(d0_ref[...], d1_ref[...])
  pre = ((p0_ref[...] + p1_ref[...]) * dis + xw1_ref[...] * deginv
         + b1_ref[...])
  h = jnp.maximum(pre, 0.0)
  h_ref[...] = h
  xw2 = jnp.dot(h, w2_ref[...], preferred_element_type=jnp.float32)
  xw2_ref[...] = xw2
  y2_ref[...] = xw2 * dis


def _tc3_body(q0_ref, q1_ref, xw2_ref, d0_ref, d1_ref, b2_ref, out_ref):
  dis, deginv = _deg_terms(d0_ref[...], d1_ref[...])
  out_ref[...] = ((q0_ref[...] + q1_ref[...]) * dis
                  + xw2_ref[...] * deginv + b2_ref[...])


_tc1 = pl.pallas_call(
    _tc1_body,
    grid=_GRID,
    in_specs=[_row_spec(D), _full_spec(D, D), _row_spec(D), _row_spec(D, _NB)],
    out_specs=[_row_spec(D), _row_spec(D)],
    out_shape=[jax.ShapeDtypeStruct((NP, D), jnp.float32)] * 2,
)

_tc2 = pl.pallas_call(
    _tc2_body,
    grid=_GRID,
    in_specs=[_row_spec(D), _row_spec(D, _NB), _row_spec(D), _row_spec(D),
              _row_spec(D, _NB), _full_spec(1, D), _full_spec(D, D)],
    out_specs=[_row_spec(D), _row_spec(D), _row_spec(D)],
    out_shape=[jax.ShapeDtypeStruct((NP, D), jnp.float32)] * 3,
)

_tc3 = pl.pallas_call(
    _tc3_body,
    grid=_GRID,
    in_specs=[_row_spec(D), _row_spec(D, _NB), _row_spec(D), _row_spec(D),
              _row_spec(D, _NB), _full_spec(1, D)],
    out_specs=_row_spec(D),
    out_shape=jax.ShapeDtypeStruct((NP, D), jnp.float32),
)


def kernel(x, edge_index, W1, b1, W2, b2):
  pad_iota = jnp.arange(EP - E, dtype=jnp.int32)
  src = jnp.concatenate([edge_index[0], pad_iota % N])
  dst = jnp.concatenate([edge_index[1], N + pad_iota % (NP - N)])
  x_p = jnp.concatenate([x, jnp.zeros((NP - N, D), jnp.float32)], axis=0)

  dp = _deg_kernel(dst)
  xw1, y1 = _tc1(x_p, W1, dp, dp)
  p = _agg_pipe(y1, src, dst)
  h, xw2, y2 = _tc2(p, p, xw1, dp, dp, b1.reshape(1, D), W2)
  q = _agg_pipe(y2, src, dst)
  logits = _tc3(q, q, xw2, dp, dp, b2.reshape(1, D))
  return (h[:N], logits[:N])


# speedup vs baseline: 4.0152x; 1.1425x over previous
import functools

import jax
import jax.numpy as jnp
from jax import lax
from jax.experimental import pallas as pl
from jax.experimental.pallas import tpu as pltpu
from jax.experimental.pallas import tpu_sc as plsc

N = 10000
NP = 10240
E = 320000
D = 128

NC = 2
NS = 16
NW = NC * NS

B = 80
K = 125
EW = B * K
S = NP // NS
SA = N // NS
NBUF = 3
KM = 123

_mesh = plsc.VectorSubcoreMesh(core_axis_name="c", subcore_axis_name="s")
_sc_params = pltpu.CompilerParams(use_tc_tiling_on_sc=False)


def _fill_rows(buf, nrows, ncols, vec):
  def body(i, _):
    for j in range(ncols // 16):
      buf[i, pl.ds(j * 16, 16)] = vec
    return 0

  lax.fori_loop(0, nrows, body, 0)


def _stage_dst(dst_hbm, didx, base, isem):
  def start(j, _):
    pltpu.make_async_copy(dst_hbm.at[pl.ds(base + j * B, B)], didx.at[j],
                          isem).start()
    return 0

  lax.fori_loop(0, K, start, 0)

  def drain(j, _):
    pltpu.make_async_copy(dst_hbm.at[pl.ds(base + j * B, B)], didx.at[j],
                          isem).wait()
    return 0

  lax.fori_loop(0, K, drain, 0)


@functools.partial(
    pl.kernel,
    out_type=jax.ShapeDtypeStruct((NC * NP, D), jnp.float32),
    mesh=_mesh,
    scratch_types=[
        pltpu.VMEM((K, B), jnp.int32),
        pltpu.VMEM((B, 16), jnp.float32),
        pltpu.VMEM((S, 16), jnp.float32),
        pltpu.VMEM((S, D), jnp.float32),
        pltpu.VMEM_SHARED((NP, 16), jnp.float32),
        pltpu.SemaphoreType.DMA,
        pltpu.SemaphoreType.DMA,
    ],
    compiler_params=_sc_params,
)
def _deg_kernel(dst_hbm, out_hbm, didx, ones_v, buf16, buf128, acc, isem,
                ssem):
  c = lax.axis_index("c")
  s = lax.axis_index("s")
  wid = c * NS + s

  _fill_rows(ones_v, B, 16, jnp.ones((16,), jnp.float32))
  _fill_rows(buf16, S, 16, jnp.zeros((16,), jnp.float32))
  _fill_rows(buf128, S, D, jnp.zeros((16,), jnp.float32))

  pltpu.sync_copy(buf16, acc.at[pl.ds(s * S, S)])

  _stage_dst(dst_hbm, didx, wid * EW, isem)
  plsc.subcore_barrier()

  def fire(j, _):
    pltpu.async_copy(ones_v, acc.at[didx.at[j]], ssem, add=True)
    return 0

  lax.fori_loop(0, K, fire, 0)

  def drain(j, _):
    pltpu.make_async_copy(ones_v, acc.at[didx.at[j]], ssem).wait()
    return 0

  lax.fori_loop(0, K, drain, 0)
  plsc.subcore_barrier()

  pltpu.sync_copy(acc.at[pl.ds(s * S, S)], buf16)

  def widen(i, _):
    buf128[i, pl.ds(0, 16)] = buf16[i, :]
    return 0

  lax.fori_loop(0, S, widen, 0)
  pltpu.sync_copy(buf128, out_hbm.at[pl.ds(c * NP + s * S, S)])


@functools.partial(
    pl.kernel,
    out_type=jax.ShapeDtypeStruct((NC * NP, D), jnp.float32),
    mesh=_mesh,
    scratch_types=[
        pltpu.VMEM((EW,), jnp.int32),
        pltpu.VMEM((K, B), jnp.int32),
        pltpu.VMEM((NBUF, B, D), jnp.float32),
        pltpu.VMEM_SHARED((N, D), jnp.float32),
        pltpu.SemaphoreType.DMA,
        [pltpu.SemaphoreType.DMA] * NBUF,
        [pltpu.SemaphoreType.DMA] * NBUF,
    ],
    compiler_params=_sc_params,
)
def _agg_kernel(y_hbm, src_hbm, dst_hbm, out_hbm, sidx, didx, rows, acc,
                isem, gsems, ssems):
  c = lax.axis_index("c")
  s = lax.axis_index("s")
  wid = c * NS + s

  def gdesc(j, b):
    return pltpu.make_async_copy(y_hbm.at[sidx.at[pl.ds(j * B, B)]],
                                 rows.at[b], gsems[b])

  def sdesc(j, b):
    return pltpu.make_async_copy(rows.at[b], acc.at[didx.at[j]], ssems[b])

  def scat(j, b):
    pltpu.async_copy(rows.at[b], acc.at[didx.at[j]], ssems[b], add=True)

  _fill_rows(rows.at[0], B, D, jnp.zeros((16,), jnp.float32))
  for k in range(SA // B):
    pltpu.sync_copy(rows.at[0], acc.at[pl.ds(s * SA + k * B, B)])
  pltpu.sync_copy(rows.at[0], acc.at[pl.ds(s * SA + SA - B, B)])

  pltpu.sync_copy(src_hbm.at[pl.ds(wid * EW, EW)], sidx)
  _stage_dst(dst_hbm, didx, wid * EW, isem)
  plsc.subcore_barrier()

  gdesc(0, 0).start()

  def outer(i, _):
    for b in range(NBUF):
      j = i * NBUF + b
      nb = (b + 1) % NBUF

      @pl.when(j >= NBUF - 1)
      def _():
        sdesc(j + 1 - NBUF, nb).wait()

      gdesc(j + 1, nb).start()
      gdesc(j, b).wait()
      scat(j, b)
    return 0

  lax.fori_loop(0, KM // NBUF, outer, 0)

  sdesc(KM - 2, 1).wait()
  gdesc(KM + 1, 1).start()
  gdesc(KM, 0).wait()
  scat(KM, 0)
  gdesc(KM + 1, 1).wait()
  scat(KM + 1, 1)
  sdesc(KM - 1, 2).wait()
  sdesc(KM, 0).wait()
  sdesc(KM + 1, 1).wait()
  plsc.subcore_barrier()

  pltpu.sync_copy(acc.at[pl.ds(s * SA, SA)],
                  out_hbm.at[pl.ds(c * NP + s * SA, SA)])


RB = 1280
_GRID = (NP // RB,)
_NB = NP // RB


def _row_spec(cols, off=0):
  return pl.BlockSpec((RB, cols), lambda i, o=off: (i + o, 0))


def _full_spec(r, c):
  return pl.BlockSpec((r, c), lambda i: (0, 0))


def _deg_terms(d0, d1):
  deg = 1.0 + d0[:, 0:1] + d1[:, 0:1]
  dis = lax.rsqrt(deg)
  return dis, 1.0 / deg


def _tc1_body(x_ref, w1_ref, d0_ref, d1_ref, xw_ref, y_ref):
  dis, _ = _deg_terms(d0_ref[...], d1_ref[...])
  xw = jnp.dot(x_ref[...], w1_ref[...], preferred_element_type=jnp.float32)
  xw_ref[...] = xw
  y_ref[...] = xw * dis


def _tc2_body(p0_ref, p1_ref, xw1_ref, d0_ref, d1_ref, b1_ref, w2_ref,
              h_ref, xw2_ref, y2_ref):
  dis, deginv = _deg_terms(d0_ref[...], d1_ref[...])
  pre = ((p0_ref[...] + p1_ref[...]) * dis + xw1_ref[...] * deginv
         + b1_ref[...])
  h = jnp.maximum(pre, 0.0)
  h_ref[...] = h
  xw2 = jnp.dot(h, w2_ref[...], preferred_element_type=jnp.float32)
  xw2_ref[...] = xw2
  y2_ref[...] = xw2 * dis


def _tc3_body(q0_ref, q1_ref, xw2_ref, d0_ref, d1_ref, b2_ref, out_ref):
  dis, deginv = _deg_terms(d0_ref[...], d1_ref[...])
  out_ref[...] = ((q0_ref[...] + q1_ref[...]) * dis
                  + xw2_ref[...] * deginv + b2_ref[...])


_tc1 = pl.pallas_call(
    _tc1_body,
    grid=_GRID,
    in_specs=[_row_spec(D), _full_spec(D, D), _row_spec(D), _row_spec(D, _NB)],
    out_specs=[_row_spec(D), _row_spec(D)],
    out_shape=[jax.ShapeDtypeStruct((NP, D), jnp.float32)] * 2,
)

_tc2 = pl.pallas_call(
    _tc2_body,
    grid=_GRID,
    in_specs=[_row_spec(D), _row_spec(D, _NB), _row_spec(D), _row_spec(D),
              _row_spec(D, _NB), _full_spec(1, D), _full_spec(D, D)],
    out_specs=[_row_spec(D), _row_spec(D), _row_spec(D)],
    out_shape=[jax.ShapeDtypeStruct((NP, D), jnp.float32)] * 3,
)

_tc3 = pl.pallas_call(
    _tc3_body,
    grid=_GRID,
    in_specs=[_row_spec(D), _row_spec(D, _NB), _row_spec(D), _row_spec(D),
              _row_spec(D, _NB), _full_spec(1, D)],
    out_specs=_row_spec(D),
    out_shape=jax.ShapeDtypeStruct((NP, D), jnp.float32),
)


def kernel(x, edge_index, W1, b1, W2, b2):
  src = edge_index[0]
  dst = edge_index[1]
  x_p = jnp.concatenate([x, jnp.zeros((NP - N, D), jnp.float32)], axis=0)

  dp = _deg_kernel(dst)
  xw1, y1 = _tc1(x_p, W1, dp, dp)
  p = _agg_kernel(y1, src, dst)
  h, xw2, y2 = _tc2(p, p, xw1, dp, dp, b1.reshape(1, D), W2)
  q = _agg_kernel(y2, src, dst)
  logits = _tc3(q, q, xw2, dp, dp, b2.reshape(1, D))
  return (h[:N], logits[:N])


# speedup vs baseline: 4.0613x; 1.0115x over previous
import functools

import jax
import jax.numpy as jnp
from jax import lax
from jax.experimental import pallas as pl
from jax.experimental.pallas import tpu as pltpu
from jax.experimental.pallas import tpu_sc as plsc

N = 10000
E = 320000
D = 128

NC = 2
NS = 16
NW = NC * NS

B = 80
K = 125
EW = B * K
SA = N // NS
NBUF = 3
KM = 123

_mesh = plsc.VectorSubcoreMesh(core_axis_name="c", subcore_axis_name="s")
_sc_params = pltpu.CompilerParams(use_tc_tiling_on_sc=False)


def _fill_rows(buf, nrows, ncols, vec):
  def body(i, _):
    for j in range(ncols // 16):
      buf[i, pl.ds(j * 16, 16)] = vec
    return 0

  lax.fori_loop(0, nrows, body, 0)


def _stage_dst(dst_hbm, didx, base, isem):
  def start(j, _):
    pltpu.make_async_copy(dst_hbm.at[pl.ds(base + j * B, B)], didx.at[j],
                          isem).start()
    return 0

  lax.fori_loop(0, K, start, 0)

  def drain(j, _):
    pltpu.make_async_copy(dst_hbm.at[pl.ds(base + j * B, B)], didx.at[j],
                          isem).wait()
    return 0

  lax.fori_loop(0, K, drain, 0)


@functools.partial(
    pl.kernel,
    out_type=jax.ShapeDtypeStruct((NC * N, D), jnp.float32),
    mesh=_mesh,
    scratch_types=[
        pltpu.VMEM((K, B), jnp.int32),
        pltpu.VMEM((B, 16), jnp.float32),
        pltpu.VMEM((SA, 16), jnp.float32),
        pltpu.VMEM((SA, D), jnp.float32),
        pltpu.VMEM_SHARED((N, 16), jnp.float32),
        pltpu.SemaphoreType.DMA,
        pltpu.SemaphoreType.DMA,
    ],
    compiler_params=_sc_params,
)
def _deg_kernel(dst_hbm, out_hbm, didx, ones_v, buf16, buf128, acc, isem,
                ssem):
  c = lax.axis_index("c")
  s = lax.axis_index("s")
  wid = c * NS + s

  _fill_rows(ones_v, B, 16, jnp.ones((16,), jnp.float32))
  _fill_rows(buf16, SA, 16, jnp.zeros((16,), jnp.float32))
  _fill_rows(buf128, SA, D, jnp.zeros((16,), jnp.float32))

  pltpu.sync_copy(buf16, acc.at[pl.ds(s * SA, SA)])

  _stage_dst(dst_hbm, didx, wid * EW, isem)
  plsc.subcore_barrier()

  def fire(j, _):
    pltpu.async_copy(ones_v, acc.at[didx.at[j]], ssem, add=True)
    return 0

  lax.fori_loop(0, K, fire, 0)

  def drain(j, _):
    pltpu.make_async_copy(ones_v, acc.at[didx.at[j]], ssem).wait()
    return 0

  lax.fori_loop(0, K, drain, 0)
  plsc.subcore_barrier()

  pltpu.sync_copy(acc.at[pl.ds(s * SA, SA)], buf16)

  def widen(i, _):
    buf128[i, pl.ds(0, 16)] = buf16[i, :]
    return 0

  lax.fori_loop(0, SA, widen, 0)
  pltpu.sync_copy(buf128, out_hbm.at[pl.ds(c * N + s * SA, SA)])


@functools.partial(
    pl.kernel,
    out_type=jax.ShapeDtypeStruct((NC * N, D), jnp.float32),
    mesh=_mesh,
    scratch_types=[
        pltpu.VMEM((EW,), jnp.int32),
        pltpu.VMEM((K, B), jnp.int32),
        pltpu.VMEM((NBUF, B, D), jnp.float32),
        pltpu.VMEM_SHARED((N, D), jnp.float32),
        pltpu.SemaphoreType.DMA,
        [pltpu.SemaphoreType.DMA] * NBUF,
        [pltpu.SemaphoreType.DMA] * NBUF,
    ],
    compiler_params=_sc_params,
)
def _agg_kernel(y_hbm, src_hbm, dst_hbm, out_hbm, sidx, didx, rows, acc,
                isem, gsems, ssems):
  c = lax.axis_index("c")
  s = lax.axis_index("s")
  wid = c * NS + s

  def gdesc(j, b):
    return pltpu.make_async_copy(y_hbm.at[sidx.at[pl.ds(j * B, B)]],
                                 rows.at[b], gsems[b])

  def sdesc(j, b):
    return pltpu.make_async_copy(rows.at[b], acc.at[didx.at[j]], ssems[b])

  def scat(j, b):
    pltpu.async_copy(rows.at[b], acc.at[didx.at[j]], ssems[b], add=True)

  _fill_rows(rows.at[0], B, D, jnp.zeros((16,), jnp.float32))
  for k in range(SA // B):
    pltpu.sync_copy(rows.at[0], acc.at[pl.ds(s * SA + k * B, B)])
  pltpu.sync_copy(rows.at[0], acc.at[pl.ds(s * SA + SA - B, B)])

  pltpu.sync_copy(src_hbm.at[pl.ds(wid * EW, EW)], sidx)
  _stage_dst(dst_hbm, didx, wid * EW, isem)
  plsc.subcore_barrier()

  gdesc(0, 0).start()

  def outer(i, _):
    for b in range(NBUF):
      j = i * NBUF + b
      nb = (b + 1) % NBUF

      @pl.when(j >= NBUF - 1)
      def _():
        sdesc(j + 1 - NBUF, nb).wait()

      gdesc(j + 1, nb).start()
      gdesc(j, b).wait()
      scat(j, b)
    return 0

  lax.fori_loop(0, KM // NBUF, outer, 0)

  sdesc(KM - 2, 1).wait()
  gdesc(KM + 1, 1).start()
  gdesc(KM, 0).wait()
  scat(KM, 0)
  gdesc(KM + 1, 1).wait()
  scat(KM + 1, 1)
  sdesc(KM - 1, 2).wait()
  sdesc(KM, 0).wait()
  sdesc(KM + 1, 1).wait()
  plsc.subcore_barrier()

  pltpu.sync_copy(acc.at[pl.ds(s * SA, SA)],
                  out_hbm.at[pl.ds(c * N + s * SA, SA)])


RB = 1000
_GRID = (N // RB,)
_NB = N // RB


def _row_spec(cols, off=0):
  return pl.BlockSpec((RB, cols), lambda i, o=off: (i + o, 0))


def _full_spec(r, c):
  return pl.BlockSpec((r, c), lambda i: (0, 0))


def _deg_terms(d0, d1):
  deg = 1.0 + d0[:, 0:1] + d1[:, 0:1]
  dis = lax.rsqrt(deg)
  return dis, 1.0 / deg


def _tc1_body(x_ref, w1_ref, d0_ref, d1_ref, xw_ref, y_ref):
  dis, _ = _deg_terms(d0_ref[...], d1_ref[...])
  xw = jnp.dot(x_ref[...], w1_ref[...], preferred_element_type=jnp.float32)
  xw_ref[...] = xw
  y_ref[...] = xw * dis


def _tc2_body(p0_ref, p1_ref, xw1_ref, d0_ref, d1_ref, b1_ref, w2_ref,
              h_ref, xw2_ref, y2_ref):
  dis, deginv = _deg_terms(d0_ref[...], d1_ref[...])
  pre = ((p0_ref[...] + p1_ref[...]) * dis + xw1_ref[...] * deginv
         + b1_ref[...])
  h = jnp.maximum(pre, 0.0)
  h_ref[...] = h
  xw2 = jnp.dot(h, w2_ref[...], preferred_element_type=jnp.float32)
  xw2_ref[...] = xw2
  y2_ref[...] = xw2 * dis


def _tc3_body(q0_ref, q1_ref, xw2_ref, d0_ref, d1_ref, b2_ref, out_ref):
  dis, deginv = _deg_terms(d0_ref[...], d1_ref[...])
  out_ref[...] = ((q0_ref[...] + q1_ref[...]) * dis
                  + xw2_ref[...] * deginv + b2_ref[...])


_tc1 = pl.pallas_call(
    _tc1_body,
    grid=_GRID,
    in_specs=[_row_spec(D), _full_spec(D, D), _row_spec(D), _row_spec(D, _NB)],
    out_specs=[_row_spec(D), _row_spec(D)],
    out_shape=[jax.ShapeDtypeStruct((N, D), jnp.float32)] * 2,
)

_tc2 = pl.pallas_call(
    _tc2_body,
    grid=_GRID,
    in_specs=[_row_spec(D), _row_spec(D, _NB), _row_spec(D), _row_spec(D),
              _row_spec(D, _NB), _full_spec(1, D), _full_spec(D, D)],
    out_specs=[_row_spec(D), _row_spec(D), _row_spec(D)],
    out_shape=[jax.ShapeDtypeStruct((N, D), jnp.float32)] * 3,
)

_tc3 = pl.pallas_call(
    _tc3_body,
    grid=_GRID,
    in_specs=[_row_spec(D), _row_spec(D, _NB), _row_spec(D), _row_spec(D),
              _row_spec(D, _NB), _full_spec(1, D)],
    out_specs=_row_spec(D),
    out_shape=jax.ShapeDtypeStruct((N, D), jnp.float32),
)


def kernel(x, edge_index, W1, b1, W2, b2):
  src = edge_index[0]
  dst = edge_index[1]

  dp = _deg_kernel(dst)
  xw1, y1 = _tc1(x, W1, dp, dp)
  p = _agg_kernel(y1, src, dst)
  h, xw2, y2 = _tc2(p, p, xw1, dp, dp, b1.reshape(1, D), W2)
  q = _agg_kernel(y2, src, dst)
  logits = _tc3(q, q, xw2, dp, dp, b2.reshape(1, D))
  return (h, logits)
